# baseline (speedup 1.0000x reference)
import jax
import jax.numpy as jnp
from jax import lax
from jax.experimental import pallas as pl
from jax.experimental.pallas import tpu as pltpu
from jax.experimental.pallas import tpu_sc as plsc

_VOCABS = [1000000, 1000000] + [100000] * 8 + [1000] * 16
_EMBED_DIM = 32
_BATCH = 16384
_NUM_FEATS = len(_VOCABS)
_OUT_W = _NUM_FEATS * _EMBED_DIM

_info = plsc.get_sparse_core_info()
_NC, _NS = _info.num_cores, _info.num_subcores
_NW = _NC * _NS
_BPW = _BATCH // _NW
_CHUNK = 64
_NCHUNK = _BPW // _CHUNK
_IDX_ROWS = _NUM_FEATS * _BPW // 128


def _body(*refs):
    tables = refs[:_NUM_FEATS]
    idx_hbm = refs[_NUM_FEATS]
    drain_hbm = refs[_NUM_FEATS + 1]
    out = refs[_NUM_FEATS + 2]
    idx_v, vbuf, drain_v, gsem0, gsem1, osem = refs[_NUM_FEATS + 3:]

    wid = lax.axis_index("s") * _NC + lax.axis_index("c")
    base = wid * _BPW
    gsems = (gsem0, gsem1)

    pltpu.sync_copy(idx_hbm.at[wid], idx_v)

    def drain(sem):
        pltpu.make_async_copy(drain_hbm, drain_v, sem).wait()

    @pl.loop(0, _NCHUNK)
    def _chunk(c):
        row0 = c * _CHUNK
        for f in range(_NUM_FEATS):
            half = 8 * f + c
            irow = half // 2
            icol = (half % 2) * _CHUNK
            sem = gsems[f % 2]

            @pl.loop(0, _CHUNK // 16)
            def _grp(g):
                iv = idx_v[irow, pl.ds(icol + g * 16, 16)]
                for j in range(16):
                    r = iv[j]
                    pltpu.async_copy(
                        tables[f].at[r],
                        vbuf.at[g * 16 + j,
                                pl.ds(f * _EMBED_DIM, _EMBED_DIM)],
                        sem,
                    )

            if f > 0:
                drain(gsems[(f - 1) % 2])
        drain(gsems[(_NUM_FEATS - 1) % 2])
        pltpu.async_copy(vbuf, out.at[pl.ds(base + row0, _CHUNK)], osem).wait()


@jax.jit
def _encoder(tables, idx_all, drain_src):
    grid_kernel = pl.kernel(
        _body,
        out_type=jax.ShapeDtypeStruct((_BATCH, _OUT_W), jnp.float32),
        mesh=plsc.VectorSubcoreMesh(core_axis_name="c", subcore_axis_name="s"),
        scratch_types=[
            pltpu.VMEM((_IDX_ROWS, 128), jnp.int32),
            pltpu.VMEM((_CHUNK, _OUT_W), jnp.float32),
            pltpu.VMEM((16, 128), jnp.float32),
            pltpu.SemaphoreType.DMA,
            pltpu.SemaphoreType.DMA,
            pltpu.SemaphoreType.DMA,
        ],
    )
    return grid_kernel(*tables, idx_all, drain_src)


def kernel(table_0, table_1, table_2, table_3, table_4, table_5, table_6,
           table_7, table_8, table_9, table_10, table_11, table_12, table_13,
           table_14, table_15, table_16, table_17, table_18, table_19,
           table_20, table_21, table_22, table_23, table_24, table_25,
           idx_0, idx_1, idx_2, idx_3, idx_4, idx_5, idx_6, idx_7, idx_8,
           idx_9, idx_10, idx_11, idx_12, idx_13, idx_14, idx_15, idx_16,
           idx_17, idx_18, idx_19, idx_20, idx_21, idx_22, idx_23, idx_24,
           idx_25):
    tables = (table_0, table_1, table_2, table_3, table_4, table_5, table_6,
              table_7, table_8, table_9, table_10, table_11, table_12,
              table_13, table_14, table_15, table_16, table_17, table_18,
              table_19, table_20, table_21, table_22, table_23, table_24,
              table_25)
    idxs = (idx_0, idx_1, idx_2, idx_3, idx_4, idx_5, idx_6, idx_7, idx_8,
            idx_9, idx_10, idx_11, idx_12, idx_13, idx_14, idx_15, idx_16,
            idx_17, idx_18, idx_19, idx_20, idx_21, idx_22, idx_23, idx_24,
            idx_25)
    idx_all = jnp.transpose(
        jnp.stack(idxs).reshape(_NUM_FEATS, _NW, _BPW), (1, 0, 2)
    ).reshape(_NW, _IDX_ROWS, 128)
    drain_src = jnp.zeros((16, 128), jnp.float32)
    return _encoder(tables, idx_all, drain_src)
